# baseline (speedup 1.0000x reference)
import functools

import jax
import jax.numpy as jnp
import numpy as np
from jax import lax
from jax.experimental import pallas as pl
from jax.experimental.pallas import tpu as pltpu
from jax.experimental.pallas import tpu_sc as plsc

N = 10000
E = 320000
F_IN = 128
HID = 32
NCLS = 2
D2 = 8
NPAD = 10240
NC, NS = 2, 16
NW = NC * NS
LANES = 128
EPAD = ((E + NW * LANES * 8 - 1) // (NW * LANES * 8)) * (NW * LANES * 8)
ROWS = EPAD // LANES
RPT = ROWS // NW
SL = NPAD // NS

_mesh = plsc.VectorSubcoreMesh(core_axis_name="c", subcore_axis_name="s")

_FILL = np.stack([np.arange(EPAD - E, dtype=np.int32) % N,
                  N + (np.arange(EPAD - E, dtype=np.int32) % (NPAD - N))])


@functools.partial(
    pl.kernel,
    out_type=jax.ShapeDtypeStruct((NC, NPAD), jnp.float32),
    mesh=_mesh,
    compiler_params=pltpu.CompilerParams(use_tc_tiling_on_sc=False),
    scratch_types=[
        pltpu.VMEM((RPT, LANES), jnp.int32),
        pltpu.VMEM((LANES,), jnp.float32),
        pltpu.VMEM_SHARED((NPAD,), jnp.float32),
        pltpu.SemaphoreType.DMA,
    ],
)
def _deg_kernel(ei_hbm, zeros_hbm, ones_hbm, out_hbm, idxc_v, ones_v, shared, dsem):
    c = lax.axis_index("c")
    s = lax.axis_index("s")
    wid = s * NC + c
    pltpu.sync_copy(ones_hbm, ones_v)
    pltpu.sync_copy(zeros_hbm, shared.at[pl.ds(s * SL, SL)])
    pltpu.sync_copy(ei_hbm.at[1, pl.ds(wid * RPT, RPT)], idxc_v)
    plsc.subcore_barrier()

    def body(j, carry):
        pltpu.async_copy(ones_v, shared.at[idxc_v.at[j]], dsem, add=True)
        return carry

    lax.fori_loop(0, RPT, body, 0)

    def dbody(j, carry):
        pltpu.make_async_copy(ones_v, shared.at[idxc_v.at[j]], dsem).wait()
        return carry

    lax.fori_loop(0, RPT, dbody, 0)
    plsc.subcore_barrier()
    pltpu.sync_copy(shared.at[pl.ds(s * SL, SL)], out_hbm.at[c, pl.ds(s * SL, SL)])


def _make_scatter(D, GRP):
    NG = RPT // GRP
    @functools.partial(
        pl.kernel,
        out_type=jax.ShapeDtypeStruct((NC, NPAD, D), jnp.float32),
        mesh=_mesh,
        compiler_params=pltpu.CompilerParams(use_tc_tiling_on_sc=False),
        scratch_types=[
            pltpu.VMEM((RPT, LANES), jnp.int32),
            pltpu.VMEM((RPT, LANES), jnp.int32),
            pltpu.VMEM((GRP, LANES, D), jnp.float32),
            pltpu.VMEM((GRP, LANES, D), jnp.float32),
            pltpu.VMEM_SHARED((NPAD, D), jnp.float32),
            pltpu.SemaphoreType.DMA,
            pltpu.SemaphoreType.DMA,
            pltpu.SemaphoreType.DMA,
            pltpu.SemaphoreType.DMA,
        ],
    )
    def _scat(ei_hbm, table_hbm, zeros_hbm, out_hbm,
              idxr_v, idxc_v, bufa_v, bufb_v, shared, sema, semb, ssema, ssemb):
        c = lax.axis_index("c")
        s = lax.axis_index("s")
        wid = s * NC + c
        pltpu.sync_copy(zeros_hbm, shared.at[pl.ds(s * SL, SL)])
        pltpu.sync_copy(ei_hbm.at[0, pl.ds(wid * RPT, RPT)], idxr_v)
        pltpu.sync_copy(ei_hbm.at[1, pl.ds(wid * RPT, RPT)], idxc_v)
        plsc.subcore_barrier()

        def fire(g, buf_v, sem):
            base = g * GRP
            for b in range(GRP):
                pltpu.async_copy(table_hbm.at[idxr_v.at[base + b]],
                                 buf_v.at[b], sem)

        def drain(g, buf_v, sem, ssem):
            base = g * GRP
            for b in range(GRP):
                pltpu.make_async_copy(table_hbm.at[idxr_v.at[base + b]],
                                      buf_v.at[b], sem).wait()
                pltpu.async_copy(buf_v.at[b], shared.at[idxc_v.at[base + b]],
                                 ssem, add=True)
            for b in range(GRP):
                pltpu.make_async_copy(buf_v.at[b],
                                      shared.at[idxc_v.at[base + b]],
                                      ssem).wait()

        fire(0, bufa_v, sema)

        def body(p, carry):
            ga = 2 * p
            fire(ga + 1, bufb_v, semb)
            drain(ga, bufa_v, sema, ssema)

            @pl.when(ga + 2 < NG)
            def _():
                fire(ga + 2, bufa_v, sema)

            drain(ga + 1, bufb_v, semb, ssemb)
            return carry

        lax.fori_loop(0, NG // 2, body, 0)
        plsc.subcore_barrier()
        pltpu.sync_copy(shared.at[pl.ds(s * SL, SL)],
                        out_hbm.at[c, pl.ds(s * SL, SL)])

    return _scat


_scat32 = _make_scatter(HID, 8)
_scat8 = _make_scatter(D2, 10)


BLK = 1000


def _tc0_body(x_ref, wi_ref, wr_ref, b_ref, h_ref, root_ref):
    x = x_ref[...]
    h_ref[...] = jnp.dot(x, wi_ref[...], preferred_element_type=jnp.float32)
    root_ref[...] = jnp.dot(x, wr_ref[...], preferred_element_type=jnp.float32) + b_ref[...]


_tc0 = pl.pallas_call(
    _tc0_body,
    out_shape=(
        jax.ShapeDtypeStruct((N, HID), jnp.float32),
        jax.ShapeDtypeStruct((N, HID), jnp.float32),
    ),
)


def _tc1_body(deg2_ref, h_ref, hs_ref, dinv_ref):
    degT = jnp.transpose(deg2_ref[...])
    deg = jnp.sum(degT[:N, :], axis=1, keepdims=True)
    dinv = jnp.where(deg > 0.0, lax.rsqrt(jnp.maximum(deg, 1e-30)), 0.0)
    hs_ref[...] = dinv * h_ref[...]
    dinv_ref[...] = dinv


_tc1 = pl.pallas_call(
    _tc1_body,
    out_shape=(
        jax.ShapeDtypeStruct((N, HID), jnp.float32),
        jax.ShapeDtypeStruct((N, 1), jnp.float32),
    ),
)


def _tc2_body(s1_ref, dinv_ref, root1_ref, w2i_ref, w2r_ref, b2_ref,
              hs2_ref, root2_ref):
    dinv = dinv_ref[...]
    agg = dinv * (s1_ref[0, :N, :] + s1_ref[1, :N, :])
    out1 = jnp.maximum(agg + root1_ref[...], 0.0)
    zpad = jnp.zeros((HID, D2 - NCLS), jnp.float32)
    w2i = jnp.concatenate([w2i_ref[...], zpad], axis=1)
    w2r = jnp.concatenate([w2r_ref[...], zpad], axis=1)
    b2 = jnp.concatenate([b2_ref[...], jnp.zeros((1, D2 - NCLS), jnp.float32)],
                         axis=1)
    h2 = jnp.dot(out1, w2i, preferred_element_type=jnp.float32)
    hs2_ref[...] = dinv * h2
    root2_ref[...] = jnp.dot(out1, w2r, preferred_element_type=jnp.float32) + b2


_tc2 = pl.pallas_call(
    _tc2_body,
    out_shape=(
        jax.ShapeDtypeStruct((N, D2), jnp.float32),
        jax.ShapeDtypeStruct((N, D2), jnp.float32),
    ),
)


def _tc3_body(s2_ref, dinv_ref, root2_ref, out_ref):
    o = dinv_ref[...] * (s2_ref[0, :N, :] + s2_ref[1, :N, :]) + root2_ref[...]
    z = o[:, 0:NCLS]
    m = jnp.max(z, axis=1, keepdims=True)
    lse = m + jnp.log(jnp.sum(jnp.exp(z - m), axis=1, keepdims=True))
    out_ref[...] = z - lse


_tc3 = pl.pallas_call(
    _tc3_body,
    out_shape=jax.ShapeDtypeStruct((N, NCLS), jnp.float32),
)


def kernel(x, edge_index, W1_init, W1_root, b1, W2_init, W2_root, b2):
    ei = jnp.concatenate([edge_index, _FILL], axis=1).reshape(2, ROWS, LANES)
    zeros32 = jnp.zeros((SL, HID), jnp.float32)

    h1, root1 = _tc0(x, W1_init, W1_root, b1.reshape(1, HID))
    deg2 = _deg_kernel(ei, zeros32[:SL, 0],
                       jnp.ones((LANES,), jnp.float32))
    hs1, dinv = _tc1(deg2, h1)

    s1 = _scat32(ei, hs1, zeros32)
    hs2, root2 = _tc2(s1, dinv, root1, W2_init, W2_root, b2.reshape(1, NCLS))

    s2 = _scat8(ei, hs2, zeros32[:, 0:D2])
    return _tc3(s2, dinv, root2)


# speedup vs baseline: 1.0022x; 1.0022x over previous
import functools

import jax
import jax.numpy as jnp
import numpy as np
from jax import lax
from jax.experimental import pallas as pl
from jax.experimental.pallas import tpu as pltpu
from jax.experimental.pallas import tpu_sc as plsc

N = 10000
E = 320000
F_IN = 128
HID = 32
NCLS = 2
D2 = 8
NPAD = 10240
NC, NS = 2, 16
NW = NC * NS
LANES = 128
EPAD = ((E + NW * LANES * 8 - 1) // (NW * LANES * 8)) * (NW * LANES * 8)
ROWS = EPAD // LANES
RPT = ROWS // NW
SL = NPAD // NS

_mesh = plsc.VectorSubcoreMesh(core_axis_name="c", subcore_axis_name="s")

_FILL = np.stack([np.arange(EPAD - E, dtype=np.int32) % N,
                  N + (np.arange(EPAD - E, dtype=np.int32) % (NPAD - N))])


@functools.partial(
    pl.kernel,
    out_type=jax.ShapeDtypeStruct((NC, NPAD), jnp.float32),
    mesh=_mesh,
    compiler_params=pltpu.CompilerParams(use_tc_tiling_on_sc=False),
    scratch_types=[
        pltpu.VMEM((RPT, LANES), jnp.int32),
        pltpu.VMEM((LANES,), jnp.float32),
        pltpu.VMEM_SHARED((NPAD,), jnp.float32),
        pltpu.SemaphoreType.DMA,
    ],
)
def _deg_kernel(ei_hbm, zeros_hbm, ones_hbm, out_hbm, idxc_v, ones_v, shared, dsem):
    c = lax.axis_index("c")
    s = lax.axis_index("s")
    wid = s * NC + c
    pltpu.sync_copy(ones_hbm, ones_v)
    pltpu.sync_copy(zeros_hbm, shared.at[pl.ds(s * SL, SL)])
    pltpu.sync_copy(ei_hbm.at[1, pl.ds(wid * RPT, RPT)], idxc_v)
    plsc.subcore_barrier()

    def body(j, carry):
        pltpu.async_copy(ones_v, shared.at[idxc_v.at[j]], dsem, add=True)
        return carry

    lax.fori_loop(0, RPT, body, 0)

    def dbody(j, carry):
        pltpu.make_async_copy(ones_v, shared.at[idxc_v.at[j]], dsem).wait()
        return carry

    lax.fori_loop(0, RPT, dbody, 0)
    plsc.subcore_barrier()
    pltpu.sync_copy(shared.at[pl.ds(s * SL, SL)], out_hbm.at[c, pl.ds(s * SL, SL)])


def _make_scatter(D, GRP):
    NG = RPT // GRP
    @functools.partial(
        pl.kernel,
        out_type=jax.ShapeDtypeStruct((NC, NPAD, D), jnp.float32),
        mesh=_mesh,
        compiler_params=pltpu.CompilerParams(use_tc_tiling_on_sc=False),
        scratch_types=[
            pltpu.VMEM((RPT, LANES), jnp.int32),
            pltpu.VMEM((RPT, LANES), jnp.int32),
            pltpu.VMEM((GRP, LANES, D), jnp.float32),
            pltpu.VMEM((GRP, LANES, D), jnp.float32),
            pltpu.VMEM_SHARED((NPAD, D), jnp.float32),
            pltpu.SemaphoreType.DMA,
            pltpu.SemaphoreType.DMA,
            pltpu.SemaphoreType.DMA,
            pltpu.SemaphoreType.DMA,
        ],
    )
    def _scat(ei_hbm, table_hbm, zeros_hbm, out_hbm,
              idxr_v, idxc_v, bufa_v, bufb_v, shared, sema, semb, ssema, ssemb):
        c = lax.axis_index("c")
        s = lax.axis_index("s")
        wid = s * NC + c
        pltpu.sync_copy(zeros_hbm, shared.at[pl.ds(s * SL, SL)])
        pltpu.sync_copy(ei_hbm.at[0, pl.ds(wid * RPT, RPT)], idxr_v)
        pltpu.sync_copy(ei_hbm.at[1, pl.ds(wid * RPT, RPT)], idxc_v)
        plsc.subcore_barrier()

        def fire(g, buf_v, sem):
            base = g * GRP
            for b in range(GRP):
                pltpu.async_copy(table_hbm.at[idxr_v.at[base + b]],
                                 buf_v.at[b], sem)

        def drain(g, buf_v, sem, ssem):
            base = g * GRP
            for b in range(GRP):
                pltpu.make_async_copy(table_hbm.at[idxr_v.at[base + b]],
                                      buf_v.at[b], sem).wait()
                pltpu.async_copy(buf_v.at[b], shared.at[idxc_v.at[base + b]],
                                 ssem, add=True)
            for b in range(GRP):
                pltpu.make_async_copy(buf_v.at[b],
                                      shared.at[idxc_v.at[base + b]],
                                      ssem).wait()

        fire(0, bufa_v, sema)

        def body(p, carry):
            ga = 2 * p
            fire(ga + 1, bufb_v, semb)
            drain(ga, bufa_v, sema, ssema)

            @pl.when(ga + 2 < NG)
            def _():
                fire(ga + 2, bufa_v, sema)

            drain(ga + 1, bufb_v, semb, ssemb)
            return carry

        lax.fori_loop(0, NG // 2, body, 0)
        plsc.subcore_barrier()
        pltpu.sync_copy(shared.at[pl.ds(s * SL, SL)],
                        out_hbm.at[c, pl.ds(s * SL, SL)])

    return _scat


_scat32 = _make_scatter(HID, 8)
_scat8 = _make_scatter(D2, 10)


BLK = 1000


def _tc1_body(deg2_ref, x_ref, wi_ref, wr_ref, b_ref, hs_ref, root_ref, dinv_ref):
    degT = jnp.transpose(deg2_ref[...])
    deg = jnp.sum(degT[:N, :], axis=1, keepdims=True)
    dinv = jnp.where(deg > 0.0, lax.rsqrt(jnp.maximum(deg, 1e-30)), 0.0)
    x = x_ref[...]
    h = jnp.dot(x, wi_ref[...], preferred_element_type=jnp.float32)
    hs_ref[...] = dinv * h
    root_ref[...] = jnp.dot(x, wr_ref[...], preferred_element_type=jnp.float32) + b_ref[...]
    dinv_ref[...] = dinv


_tc1 = pl.pallas_call(
    _tc1_body,
    out_shape=(
        jax.ShapeDtypeStruct((N, HID), jnp.float32),
        jax.ShapeDtypeStruct((N, HID), jnp.float32),
        jax.ShapeDtypeStruct((N, 1), jnp.float32),
    ),
)


def _tc2_body(s1_ref, dinv_ref, root1_ref, w2i_ref, w2r_ref, b2_ref,
              hs2_ref, root2_ref):
    dinv = dinv_ref[...]
    agg = dinv * (s1_ref[0, :N, :] + s1_ref[1, :N, :])
    out1 = jnp.maximum(agg + root1_ref[...], 0.0)
    zpad = jnp.zeros((HID, D2 - NCLS), jnp.float32)
    w2i = jnp.concatenate([w2i_ref[...], zpad], axis=1)
    w2r = jnp.concatenate([w2r_ref[...], zpad], axis=1)
    b2 = jnp.concatenate([b2_ref[...], jnp.zeros((1, D2 - NCLS), jnp.float32)],
                         axis=1)
    h2 = jnp.dot(out1, w2i, preferred_element_type=jnp.float32)
    hs2_ref[...] = dinv * h2
    root2_ref[...] = jnp.dot(out1, w2r, preferred_element_type=jnp.float32) + b2


_tc2 = pl.pallas_call(
    _tc2_body,
    compiler_params=pltpu.CompilerParams(
        allow_input_fusion=[True, False, False, False, False, False]),
    out_shape=(
        jax.ShapeDtypeStruct((N, D2), jnp.float32),
        jax.ShapeDtypeStruct((N, D2), jnp.float32),
    ),
)


def _tc3_body(s2_ref, dinv_ref, root2_ref, out_ref):
    o = dinv_ref[...] * (s2_ref[0, :N, :] + s2_ref[1, :N, :]) + root2_ref[...]
    z = o[:, 0:NCLS]
    m = jnp.max(z, axis=1, keepdims=True)
    lse = m + jnp.log(jnp.sum(jnp.exp(z - m), axis=1, keepdims=True))
    out_ref[...] = z - lse


_tc3 = pl.pallas_call(
    _tc3_body,
    compiler_params=pltpu.CompilerParams(
        allow_input_fusion=[True, False, False]),
    out_shape=jax.ShapeDtypeStruct((N, NCLS), jnp.float32),
)


def kernel(x, edge_index, W1_init, W1_root, b1, W2_init, W2_root, b2):
    ei = jnp.concatenate([edge_index, _FILL], axis=1).reshape(2, ROWS, LANES)
    zeros32 = jnp.zeros((SL, HID), jnp.float32)

    deg2 = _deg_kernel(ei, zeros32[:SL, 0],
                       jnp.ones((LANES,), jnp.float32))
    hs1, root1, dinv = _tc1(deg2, x, W1_init, W1_root, b1.reshape(1, HID))

    s1 = _scat32(ei, hs1, zeros32)
    hs2, root2 = _tc2(s1, dinv, root1, W2_init, W2_root, b2.reshape(1, NCLS))

    s2 = _scat8(ei, hs2, zeros32[:, 0:D2])
    return _tc3(s2, dinv, root2)


# speedup vs baseline: 1.0037x; 1.0015x over previous
import functools

import jax
import jax.numpy as jnp
import numpy as np
from jax import lax
from jax.experimental import pallas as pl
from jax.experimental.pallas import tpu as pltpu
from jax.experimental.pallas import tpu_sc as plsc

N = 10000
E = 320000
F_IN = 128
HID = 32
NCLS = 2
D2 = 8
NPAD = 10240
NC, NS = 2, 16
NW = NC * NS
LANES = 128
EPAD = ((E + NW * LANES * 8 - 1) // (NW * LANES * 8)) * (NW * LANES * 8)
ROWS = EPAD // LANES
RPT = ROWS // NW
SL = NPAD // NS

_mesh = plsc.VectorSubcoreMesh(core_axis_name="c", subcore_axis_name="s")

_FILL = np.stack([np.arange(EPAD - E, dtype=np.int32) % N,
                  N + (np.arange(EPAD - E, dtype=np.int32) % (NPAD - N))])


@functools.partial(
    pl.kernel,
    out_type=jax.ShapeDtypeStruct((NC, NPAD), jnp.float32),
    mesh=_mesh,
    compiler_params=pltpu.CompilerParams(use_tc_tiling_on_sc=False),
    scratch_types=[
        pltpu.VMEM((RPT, LANES), jnp.int32),
        pltpu.VMEM((LANES,), jnp.float32),
        pltpu.VMEM_SHARED((NPAD,), jnp.float32),
        pltpu.SemaphoreType.DMA,
    ],
)
def _deg_kernel(ei_hbm, zeros_hbm, ones_hbm, out_hbm, idxc_v, ones_v, shared, dsem):
    c = lax.axis_index("c")
    s = lax.axis_index("s")
    wid = s * NC + c
    pltpu.sync_copy(ones_hbm, ones_v)
    pltpu.sync_copy(zeros_hbm, shared.at[pl.ds(s * SL, SL)])
    pltpu.sync_copy(ei_hbm.at[1, pl.ds(wid * RPT, RPT)], idxc_v)
    plsc.subcore_barrier()

    def body(j, carry):
        pltpu.async_copy(ones_v, shared.at[idxc_v.at[j]], dsem, add=True)
        return carry

    lax.fori_loop(0, RPT, body, 0)

    def dbody(j, carry):
        pltpu.make_async_copy(ones_v, shared.at[idxc_v.at[j]], dsem).wait()
        return carry

    lax.fori_loop(0, RPT, dbody, 0)
    plsc.subcore_barrier()
    pltpu.sync_copy(shared.at[pl.ds(s * SL, SL)], out_hbm.at[c, pl.ds(s * SL, SL)])


def _make_scatter(D, GRP):
    NG = RPT // GRP
    @functools.partial(
        pl.kernel,
        out_type=jax.ShapeDtypeStruct((NC, NPAD, D), jnp.float32),
        mesh=_mesh,
        compiler_params=pltpu.CompilerParams(use_tc_tiling_on_sc=False),
        scratch_types=[
            pltpu.VMEM((RPT, LANES), jnp.int32),
            pltpu.VMEM((RPT, LANES), jnp.int32),
            pltpu.VMEM((GRP, LANES, D), jnp.float32),
            pltpu.VMEM((GRP, LANES, D), jnp.float32),
            pltpu.VMEM_SHARED((NPAD, D), jnp.float32),
            pltpu.SemaphoreType.DMA,
            pltpu.SemaphoreType.DMA,
            pltpu.SemaphoreType.DMA,
            pltpu.SemaphoreType.DMA,
        ],
    )
    def _scat(ei_hbm, table_hbm, zeros_hbm, out_hbm,
              idxr_v, idxc_v, bufa_v, bufb_v, shared, sema, semb, ssema, ssemb):
        c = lax.axis_index("c")
        s = lax.axis_index("s")
        wid = s * NC + c
        pltpu.sync_copy(zeros_hbm, shared.at[pl.ds(s * SL, SL)])
        pltpu.sync_copy(ei_hbm.at[0, pl.ds(wid * RPT, RPT)], idxr_v)
        pltpu.sync_copy(ei_hbm.at[1, pl.ds(wid * RPT, RPT)], idxc_v)
        plsc.subcore_barrier()

        def fire(g, buf_v, sem):
            base = g * GRP
            for b in range(GRP):
                pltpu.async_copy(table_hbm.at[idxr_v.at[base + b]],
                                 buf_v.at[b], sem)

        def drain(g, buf_v, sem, ssem):
            base = g * GRP
            for b in range(GRP):
                pltpu.make_async_copy(table_hbm.at[idxr_v.at[base + b]],
                                      buf_v.at[b], sem).wait()
                pltpu.async_copy(buf_v.at[b], shared.at[idxc_v.at[base + b]],
                                 ssem, add=True)
            for b in range(GRP):
                pltpu.make_async_copy(buf_v.at[b],
                                      shared.at[idxc_v.at[base + b]],
                                      ssem).wait()

        fire(0, bufa_v, sema)

        def body(p, carry):
            ga = 2 * p
            fire(ga + 1, bufb_v, semb)
            drain(ga, bufa_v, sema, ssema)

            @pl.when(ga + 2 < NG)
            def _():
                fire(ga + 2, bufa_v, sema)

            drain(ga + 1, bufb_v, semb, ssemb)
            return carry

        lax.fori_loop(0, NG // 2, body, 0)
        plsc.subcore_barrier()
        pltpu.sync_copy(shared.at[pl.ds(s * SL, SL)],
                        out_hbm.at[c, pl.ds(s * SL, SL)])

    return _scat


_scat32 = _make_scatter(HID, 8)
_scat8 = _make_scatter(D2, 10)


BLK = 1000


def _tc1_body(deg2_ref, x_ref, wi_ref, wr_ref, b_ref, hs_ref, root_ref, dinv_ref):
    degT = jnp.transpose(deg2_ref[...])
    deg = jnp.sum(degT[:N, :], axis=1, keepdims=True)
    dinv = jnp.where(deg > 0.0, lax.rsqrt(jnp.maximum(deg, 1e-30)), 0.0)
    x = x_ref[...]
    h = jnp.dot(x, wi_ref[...], preferred_element_type=jnp.float32)
    hs_ref[...] = dinv * h
    root_ref[...] = jnp.dot(x, wr_ref[...], preferred_element_type=jnp.float32) + b_ref[...]
    dinv_ref[...] = dinv


_tc1 = pl.pallas_call(
    _tc1_body,
    out_shape=(
        jax.ShapeDtypeStruct((N, HID), jnp.float32),
        jax.ShapeDtypeStruct((N, HID), jnp.float32),
        jax.ShapeDtypeStruct((N, 1), jnp.float32),
    ),
)


def _tc2_body(s1_ref, dinv_ref, root1_ref, w2i_ref, w2r_ref, b2_ref,
              hs2_ref, root2_ref):
    dinv = dinv_ref[...]
    agg = dinv * (s1_ref[0, :N, :] + s1_ref[1, :N, :])
    out1 = jnp.maximum(agg + root1_ref[...], 0.0)
    zpad = jnp.zeros((HID, D2 - NCLS), jnp.float32)
    w2i = jnp.concatenate([w2i_ref[...], zpad], axis=1)
    w2r = jnp.concatenate([w2r_ref[...], zpad], axis=1)
    b2 = jnp.concatenate([b2_ref[...], jnp.zeros((1, D2 - NCLS), jnp.float32)],
                         axis=1)
    h2 = jnp.dot(out1, w2i, preferred_element_type=jnp.float32)
    hs2_ref[...] = dinv * h2
    root2_ref[...] = jnp.dot(out1, w2r, preferred_element_type=jnp.float32) + b2


_tc2 = pl.pallas_call(
    _tc2_body,
    out_shape=(
        jax.ShapeDtypeStruct((N, D2), jnp.float32),
        jax.ShapeDtypeStruct((N, D2), jnp.float32),
    ),
)


def _tc3_body(s2_ref, dinv_ref, root2_ref, out_ref):
    o = dinv_ref[...] * (s2_ref[0, :N, :] + s2_ref[1, :N, :]) + root2_ref[...]
    z = o[:, 0:NCLS]
    m = jnp.max(z, axis=1, keepdims=True)
    lse = m + jnp.log(jnp.sum(jnp.exp(z - m), axis=1, keepdims=True))
    out_ref[...] = z - lse


_tc3 = pl.pallas_call(
    _tc3_body,
    out_shape=jax.ShapeDtypeStruct((N, NCLS), jnp.float32),
)


def kernel(x, edge_index, W1_init, W1_root, b1, W2_init, W2_root, b2):
    ei = jnp.concatenate([edge_index, _FILL], axis=1).reshape(2, ROWS, LANES)
    zeros32 = jnp.zeros((SL, HID), jnp.float32)

    deg2 = _deg_kernel(ei, zeros32[:SL, 0],
                       jnp.ones((LANES,), jnp.float32))
    hs1, root1, dinv = _tc1(deg2, x, W1_init, W1_root, b1.reshape(1, HID))

    s1 = _scat32(ei, hs1, zeros32)
    hs2, root2 = _tc2(s1, dinv, root1, W2_init, W2_root, b2.reshape(1, NCLS))

    s2 = _scat8(ei, hs2, zeros32[:, 0:D2])
    return _tc3(s2, dinv, root2)
